# Initial kernel scaffold; baseline (speedup 1.0000x reference)
import jax
import jax.numpy as jnp
from jax.experimental import pallas as pl


def kernel(x, in_degree, out_degree, atom_table, in_deg_table, out_deg_table, graph_token):
    raise NotImplementedError("write your pallas kernel here")


# speedup vs baseline: 11.0362x; 11.0362x over previous
import functools

import jax
import jax.numpy as jnp
from jax import lax
from jax.experimental import pallas as pl
from jax.experimental.pallas import tpu as pltpu
from jax.experimental.pallas import tpu_sc as plsc

B, N, F = 1024, 128, 9
D = 64
NP1 = N + 1

_NC, _NS = 2, 16
NW = _NC * _NS
BPW = B // NW
IDX_PER_B = N * F

_mesh = plsc.VectorSubcoreMesh(core_axis_name="c", subcore_axis_name="s")


@functools.partial(
    pl.kernel,
    out_type=jax.ShapeDtypeStruct((B, NP1, D), jnp.float32),
    mesh=_mesh,
    compiler_params=pltpu.CompilerParams(use_tc_tiling_on_sc=False),
    scratch_types=[
        pltpu.VMEM((F, N), jnp.int32),
        pltpu.VMEM((N,), jnp.int32),
        pltpu.VMEM((N,), jnp.int32),
        pltpu.VMEM((IDX_PER_B, D), jnp.float32),
        pltpu.VMEM((N, D), jnp.float32),
        pltpu.VMEM((N, D), jnp.float32),
        pltpu.VMEM((NP1, D), jnp.float32),
        pltpu.SemaphoreType.DMA,
    ],
)
def _node_feature_sc(x_hbm, ind_hbm, outd_hbm, atom_hbm, intab_hbm,
                     outtab_hbm, gt_hbm, out_hbm,
                     idx_a, idx_i, idx_o, rows_a, rows_i, rows_o, obuf, sem):
    wid = lax.axis_index("s") * _NC + lax.axis_index("c")

    pltpu.sync_copy(gt_hbm, obuf.at[pl.ds(0, 1)])

    def batch_body(g, carry):
        b = wid * BPW + g
        pltpu.sync_copy(x_hbm.at[b], idx_a)
        pltpu.sync_copy(ind_hbm.at[b], idx_i)
        pltpu.sync_copy(outd_hbm.at[b], idx_o)
        cps = []
        for f in range(F):
            cps.append(pltpu.async_copy(
                atom_hbm.at[idx_a.at[f]], rows_a.at[pl.ds(f * N, N)], sem))
        cps.append(pltpu.async_copy(intab_hbm.at[idx_i], rows_i, sem))
        cps.append(pltpu.async_copy(outtab_hbm.at[idx_o], rows_o, sem))
        for cp in cps:
            cp.wait()

        def node_body(n, carry2):
            base = n * F
            for k in range(D // 16):
                sl = pl.ds(k * 16, 16)
                acc = rows_i[n, sl] + rows_o[n, sl]
                for f in range(F):
                    acc = acc + rows_a[base + f, sl]
                obuf[n + 1, sl] = acc
            return carry2

        lax.fori_loop(0, N, node_body, 0)
        pltpu.sync_copy(obuf, out_hbm.at[b])
        return carry

    lax.fori_loop(0, BPW, batch_body, 0)


def kernel(x, in_degree, out_degree, atom_table, in_deg_table,
           out_deg_table, graph_token):
    x3 = x.astype(jnp.int32).reshape(B, F, N)
    return _node_feature_sc(
        x3, in_degree.astype(jnp.int32), out_degree.astype(jnp.int32),
        atom_table, in_deg_table, out_deg_table, graph_token)


# speedup vs baseline: 12.5289x; 1.1353x over previous
import functools

import jax
import jax.numpy as jnp
from jax import lax
from jax.experimental import pallas as pl
from jax.experimental.pallas import tpu as pltpu
from jax.experimental.pallas import tpu_sc as plsc

B, N, F = 1024, 128, 9
D = 64
NP1 = N + 1
H = N // 2
IDX_PER_C = H * F

_NC, _NS = 2, 16
NW = _NC * _NS
BPW = B // NW

_mesh = plsc.VectorSubcoreMesh(core_axis_name="c", subcore_axis_name="s")


@functools.partial(
    pl.kernel,
    out_type=jax.ShapeDtypeStruct((B, NP1, D), jnp.float32),
    mesh=_mesh,
    compiler_params=pltpu.CompilerParams(use_tc_tiling_on_sc=False),
    scratch_types=[
        pltpu.VMEM((2, F, H), jnp.int32),
        pltpu.VMEM((2, H), jnp.int32),
        pltpu.VMEM((2, H), jnp.int32),
        pltpu.VMEM((2, IDX_PER_C, D), jnp.float32),
        pltpu.VMEM((2, H, D), jnp.float32),
        pltpu.VMEM((2, H, D), jnp.float32),
        pltpu.VMEM((1 + H, D), jnp.float32),
        pltpu.VMEM((H, D), jnp.float32),
        pltpu.SemaphoreType.DMA,
        pltpu.SemaphoreType.DMA,
        pltpu.SemaphoreType.DMA,
        pltpu.SemaphoreType.DMA,
    ],
)
def _node_feature_sc(x_hbm, ind_hbm, outd_hbm, atom_hbm, intab_hbm,
                     outtab_hbm, gt_hbm, out_hbm,
                     idx_a, idx_i, idx_o, rows_a, rows_i, rows_o,
                     obuf0, obuf1, sem_g0, sem_g1, sem_s0, sem_s1):
    wid = lax.axis_index("s") * _NC + lax.axis_index("c")
    b0 = wid * BPW
    sem_g = (sem_g0, sem_g1)

    def stage_and_fire(b, h, p):
        pltpu.sync_copy(x_hbm.at[b, h], idx_a.at[p])
        pltpu.sync_copy(ind_hbm.at[b, h], idx_i.at[p])
        pltpu.sync_copy(outd_hbm.at[b, h], idx_o.at[p])
        for j in range(F):
            pltpu.async_copy(atom_hbm.at[idx_a.at[p, j]],
                             rows_a.at[p, pl.ds(j * H, H)], sem_g[p])
        pltpu.async_copy(intab_hbm.at[idx_i.at[p]], rows_i.at[p], sem_g[p])
        pltpu.async_copy(outtab_hbm.at[idx_o.at[p]], rows_o.at[p], sem_g[p])

    def drain_gathers(p):
        pltpu.make_async_copy(atom_hbm.at[pl.ds(0, IDX_PER_C)],
                              rows_a.at[p], sem_g[p]).wait()
        pltpu.make_async_copy(intab_hbm.at[pl.ds(0, H)],
                              rows_i.at[p], sem_g[p]).wait()
        pltpu.make_async_copy(outtab_hbm.at[pl.ds(0, H)],
                              rows_o.at[p], sem_g[p]).wait()

    def compute(p, obuf, row_off):
        def node_body(n, carry):
            base = n * F
            for k in range(D // 16):
                sl = pl.ds(k * 16, 16)
                acc = rows_i[p, n, sl] + rows_o[p, n, sl]
                for f in range(F):
                    acc = acc + rows_a[p, base + f, sl]
                obuf[n + row_off, sl] = acc
            return carry
        lax.fori_loop(0, H, node_body, 0)

    pltpu.sync_copy(gt_hbm, obuf0.at[pl.ds(0, 1)])
    stage_and_fire(b0, 0, 0)

    def batch_body(g, carry):
        b = b0 + g

        stage_and_fire(b, 1, 1)
        drain_gathers(0)

        @pl.when(g >= 1)
        def _():
            pltpu.make_async_copy(atom_hbm.at[pl.ds(0, 1 + H)],
                                  obuf0, sem_s0).wait()
        compute(0, obuf0, 1)
        pltpu.async_copy(obuf0, out_hbm.at[b, pl.ds(0, 1 + H)], sem_s0)

        @pl.when(g < BPW - 1)
        def _():
            stage_and_fire(b + 1, 0, 0)
        drain_gathers(1)

        @pl.when(g >= 1)
        def _():
            pltpu.make_async_copy(atom_hbm.at[pl.ds(0, H)],
                                  obuf1, sem_s1).wait()
        compute(1, obuf1, 0)
        pltpu.async_copy(obuf1, out_hbm.at[b, pl.ds(1 + H, H)], sem_s1)
        return carry

    lax.fori_loop(0, BPW, batch_body, 0)

    pltpu.make_async_copy(atom_hbm.at[pl.ds(0, 1 + H)], obuf0, sem_s0).wait()
    pltpu.make_async_copy(atom_hbm.at[pl.ds(0, H)], obuf1, sem_s1).wait()


def kernel(x, in_degree, out_degree, atom_table, in_deg_table,
           out_deg_table, graph_token):
    x4 = x.astype(jnp.int32).reshape(B, 2, F, H)
    ind2 = in_degree.astype(jnp.int32).reshape(B, 2, H)
    outd2 = out_degree.astype(jnp.int32).reshape(B, 2, H)
    return _node_feature_sc(x4, ind2, outd2, atom_table, in_deg_table,
                            out_deg_table, graph_token)


# speedup vs baseline: 16.4194x; 1.3105x over previous
import functools

import jax
import jax.numpy as jnp
from jax import lax
from jax.experimental import pallas as pl
from jax.experimental.pallas import tpu as pltpu
from jax.experimental.pallas import tpu_sc as plsc

B, N, F = 1024, 128, 9
D = 64
NP1 = N + 1
H = N // 2
IDX_PER_C = H * F

_NC, _NS = 2, 16
NW = _NC * _NS
BPW = B // NW

_mesh = plsc.VectorSubcoreMesh(core_axis_name="c", subcore_axis_name="s")


@functools.partial(
    pl.kernel,
    out_type=jax.ShapeDtypeStruct((B, NP1, D), jnp.float32),
    mesh=_mesh,
    compiler_params=pltpu.CompilerParams(use_tc_tiling_on_sc=False),
    scratch_types=[
        pltpu.VMEM((2, F, H), jnp.int32),
        pltpu.VMEM((2, H), jnp.int32),
        pltpu.VMEM((2, H), jnp.int32),
        pltpu.VMEM((2, IDX_PER_C, D), jnp.float32),
        pltpu.VMEM((2, H, D), jnp.float32),
        pltpu.VMEM((2, H, D), jnp.float32),
        pltpu.VMEM((1 + H, D), jnp.float32),
        pltpu.VMEM((H, D), jnp.float32),
        pltpu.SemaphoreType.DMA,
        pltpu.SemaphoreType.DMA,
        pltpu.SemaphoreType.DMA,
        pltpu.SemaphoreType.DMA,
    ],
)
def _node_feature_sc(x_hbm, ind_hbm, outd_hbm, atom_hbm, intab_hbm,
                     outtab_hbm, gt_hbm, out_hbm,
                     idx_a, idx_i, idx_o, rows_a, rows_i, rows_o,
                     obuf0, obuf1, sem_g0, sem_g1, sem_s0, sem_s1):
    wid = lax.axis_index("s") * _NC + lax.axis_index("c")
    b0 = wid * BPW
    sem_g = (sem_g0, sem_g1)

    def stage_and_fire(b, h, p):
        pltpu.sync_copy(x_hbm.at[b, h], idx_a.at[p])
        pltpu.sync_copy(ind_hbm.at[b, h], idx_i.at[p])
        pltpu.sync_copy(outd_hbm.at[b, h], idx_o.at[p])
        for j in range(F):
            pltpu.async_copy(atom_hbm.at[idx_a.at[p, j]],
                             rows_a.at[p, pl.ds(j * H, H)], sem_g[p])
        pltpu.async_copy(intab_hbm.at[idx_i.at[p]], rows_i.at[p], sem_g[p])
        pltpu.async_copy(outtab_hbm.at[idx_o.at[p]], rows_o.at[p], sem_g[p])

    def drain_gathers(p):
        pltpu.make_async_copy(atom_hbm.at[pl.ds(0, IDX_PER_C)],
                              rows_a.at[p], sem_g[p]).wait()
        pltpu.make_async_copy(intab_hbm.at[pl.ds(0, H)],
                              rows_i.at[p], sem_g[p]).wait()
        pltpu.make_async_copy(outtab_hbm.at[pl.ds(0, H)],
                              rows_o.at[p], sem_g[p]).wait()

    def compute(p, obuf, row_off):
        @plsc.parallel_loop(0, H, unroll=4)
        def node_body(n):
            base = n * F
            for k in range(D // 16):
                sl = pl.ds(k * 16, 16)
                a = [rows_a[p, base + f, sl] for f in range(F)]
                t0 = a[0] + a[1]
                t1 = a[2] + a[3]
                t2 = a[4] + a[5]
                t3 = a[6] + a[7]
                t4 = rows_i[p, n, sl] + rows_o[p, n, sl]
                obuf[n + row_off, sl] = ((t0 + t1) + (t2 + t3)) + ((t4 + a[8]))

    pltpu.sync_copy(gt_hbm, obuf0.at[pl.ds(0, 1)])
    stage_and_fire(b0, 0, 0)

    def batch_body(g, carry):
        b = b0 + g

        stage_and_fire(b, 1, 1)
        drain_gathers(0)

        @pl.when(g >= 1)
        def _():
            pltpu.make_async_copy(atom_hbm.at[pl.ds(0, 1 + H)],
                                  obuf0, sem_s0).wait()
        compute(0, obuf0, 1)
        pltpu.async_copy(obuf0, out_hbm.at[b, pl.ds(0, 1 + H)], sem_s0)

        @pl.when(g < BPW - 1)
        def _():
            stage_and_fire(b + 1, 0, 0)
        drain_gathers(1)

        @pl.when(g >= 1)
        def _():
            pltpu.make_async_copy(atom_hbm.at[pl.ds(0, H)],
                                  obuf1, sem_s1).wait()
        compute(1, obuf1, 0)
        pltpu.async_copy(obuf1, out_hbm.at[b, pl.ds(1 + H, H)], sem_s1)
        return carry

    lax.fori_loop(0, BPW, batch_body, 0)

    pltpu.make_async_copy(atom_hbm.at[pl.ds(0, 1 + H)], obuf0, sem_s0).wait()
    pltpu.make_async_copy(atom_hbm.at[pl.ds(0, H)], obuf1, sem_s1).wait()


def kernel(x, in_degree, out_degree, atom_table, in_deg_table,
           out_deg_table, graph_token):
    x4 = x.astype(jnp.int32).reshape(B, 2, F, H)
    ind2 = in_degree.astype(jnp.int32).reshape(B, 2, H)
    outd2 = out_degree.astype(jnp.int32).reshape(B, 2, H)
    return _node_feature_sc(x4, ind2, outd2, atom_table, in_deg_table,
                            out_deg_table, graph_token)


# speedup vs baseline: 19.1024x; 1.1634x over previous
import functools

import jax
import jax.numpy as jnp
from jax import lax
from jax.experimental import pallas as pl
from jax.experimental.pallas import tpu as pltpu
from jax.experimental.pallas import tpu_sc as plsc

B, N, F = 1024, 128, 9
D = 64
NP1 = N + 1
H = N // 2
IDX_PER_C = H * F

_NC, _NS = 2, 16
NW = _NC * _NS
BPW = B // NW

_mesh = plsc.VectorSubcoreMesh(core_axis_name="c", subcore_axis_name="s")


@functools.partial(
    pl.kernel,
    out_type=jax.ShapeDtypeStruct((B, NP1, D), jnp.float32),
    mesh=_mesh,
    compiler_params=pltpu.CompilerParams(use_tc_tiling_on_sc=False),
    scratch_types=[
        pltpu.VMEM((2, 2, F, H), jnp.int32),
        pltpu.VMEM((2, 2, H), jnp.int32),
        pltpu.VMEM((2, 2, H), jnp.int32),
        pltpu.VMEM((2, IDX_PER_C, D), jnp.float32),
        pltpu.VMEM((2, H, D), jnp.float32),
        pltpu.VMEM((2, H, D), jnp.float32),
        pltpu.VMEM((1 + H, D), jnp.float32),
        pltpu.VMEM((H, D), jnp.float32),
        pltpu.SemaphoreType.DMA,
        pltpu.SemaphoreType.DMA,
        pltpu.SemaphoreType.DMA,
        pltpu.SemaphoreType.DMA,
        pltpu.SemaphoreType.DMA,
        pltpu.SemaphoreType.DMA,
    ],
)
def _node_feature_sc(x_hbm, ind_hbm, outd_hbm, atom_hbm, intab_hbm,
                     outtab_hbm, gt_hbm, out_hbm,
                     idx_a, idx_i, idx_o, rows_a, rows_i, rows_o,
                     obuf0, obuf1, sem_g0, sem_g1, sem_s0, sem_s1,
                     sem_i0, sem_i1):
    wid = lax.axis_index("s") * _NC + lax.axis_index("c")
    b0 = wid * BPW
    sem_g = (sem_g0, sem_g1)
    sem_i = (sem_i0, sem_i1)

    def stage_idx(b, pb, sem):
        pltpu.async_copy(x_hbm.at[b], idx_a.at[pb], sem)
        pltpu.async_copy(ind_hbm.at[b], idx_i.at[pb], sem)
        pltpu.async_copy(outd_hbm.at[b], idx_o.at[pb], sem)

    def wait_idx(pb, sem):
        pltpu.make_async_copy(x_hbm.at[0], idx_a.at[pb], sem).wait()
        pltpu.make_async_copy(ind_hbm.at[0], idx_i.at[pb], sem).wait()
        pltpu.make_async_copy(outd_hbm.at[0], idx_o.at[pb], sem).wait()

    def fire_gathers(pb, h, p):
        for j in range(F):
            pltpu.async_copy(atom_hbm.at[idx_a.at[pb, h, j]],
                             rows_a.at[p, pl.ds(j * H, H)], sem_g[p])
        pltpu.async_copy(intab_hbm.at[idx_i.at[pb, h]], rows_i.at[p],
                         sem_g[p])
        pltpu.async_copy(outtab_hbm.at[idx_o.at[pb, h]], rows_o.at[p],
                         sem_g[p])

    def drain_gathers(p):
        pltpu.make_async_copy(atom_hbm.at[pl.ds(0, IDX_PER_C)],
                              rows_a.at[p], sem_g[p]).wait()
        pltpu.make_async_copy(intab_hbm.at[pl.ds(0, H)],
                              rows_i.at[p], sem_g[p]).wait()
        pltpu.make_async_copy(outtab_hbm.at[pl.ds(0, H)],
                              rows_o.at[p], sem_g[p]).wait()

    def compute(p, obuf, row_off):
        @plsc.parallel_loop(0, H, unroll=4)
        def node_body(n):
            base = n * F
            for k in range(D // 16):
                sl = pl.ds(k * 16, 16)
                a = [rows_a[p, base + f, sl] for f in range(F)]
                t0 = a[0] + a[1]
                t1 = a[2] + a[3]
                t2 = a[4] + a[5]
                t3 = a[6] + a[7]
                t4 = rows_i[p, n, sl] + rows_o[p, n, sl]
                obuf[n + row_off, sl] = ((t0 + t1) + (t2 + t3)) + (t4 + a[8])

    pltpu.sync_copy(gt_hbm, obuf0.at[pl.ds(0, 1)])
    stage_idx(b0, 0, sem_i[0])
    wait_idx(0, sem_i[0])
    fire_gathers(0, 0, 0)

    def batch_body(g, carry):
        b = b0 + g
        pb = lax.rem(g, 2)

        @pl.when(g < BPW - 1)
        def _():
            stage_idx(b + 1, 1 - pb, sem_i[1])

        fire_gathers(pb, 1, 1)
        drain_gathers(0)

        @pl.when(g >= 1)
        def _():
            pltpu.make_async_copy(atom_hbm.at[pl.ds(0, 1 + H)],
                                  obuf0, sem_s0).wait()
        compute(0, obuf0, 1)
        pltpu.async_copy(obuf0, out_hbm.at[b, pl.ds(0, 1 + H)], sem_s0)

        @pl.when(g < BPW - 1)
        def _():
            wait_idx(1 - pb, sem_i[1])
            fire_gathers(1 - pb, 0, 0)
        drain_gathers(1)

        @pl.when(g >= 1)
        def _():
            pltpu.make_async_copy(atom_hbm.at[pl.ds(0, H)],
                                  obuf1, sem_s1).wait()
        compute(1, obuf1, 0)
        pltpu.async_copy(obuf1, out_hbm.at[b, pl.ds(1 + H, H)], sem_s1)
        return carry

    lax.fori_loop(0, BPW, batch_body, 0)

    pltpu.make_async_copy(atom_hbm.at[pl.ds(0, 1 + H)], obuf0, sem_s0).wait()
    pltpu.make_async_copy(atom_hbm.at[pl.ds(0, H)], obuf1, sem_s1).wait()


def kernel(x, in_degree, out_degree, atom_table, in_deg_table,
           out_deg_table, graph_token):
    x4 = x.astype(jnp.int32).reshape(B, 2, F, H)
    ind2 = in_degree.astype(jnp.int32).reshape(B, 2, H)
    outd2 = out_degree.astype(jnp.int32).reshape(B, 2, H)
    return _node_feature_sc(x4, ind2, outd2, atom_table, in_deg_table,
                            out_deg_table, graph_token)
